# baseline (speedup 1.0000x reference)
import functools

import jax
import jax.numpy as jnp
from jax import lax
from jax.experimental import pallas as pl
from jax.experimental.pallas import tpu as pltpu
from jax.experimental.pallas import tpu_sc as plsc

NUM_EXPERTS = 16
TOP_K = 2
D_MODEL = 768
T = 2048
BLK = 128
NV = T * TOP_K // BLK + NUM_EXPERTS - 1 + 1
XPAD = NV * BLK
NW = 32
TPW = T // NW
LANES = 16
NBAND = 8
BAND = T // NBAND


def _routing_kernel(x_ref, rw_ref, rb_ref, eb_ref,
                    d1_ref, d2_ref, g1_ref, g2_ref, bias_ref, cnt_ref):
    x = x_ref[...]
    logits = jnp.dot(x, rw_ref[...], preferred_element_type=jnp.float32)
    logits = logits + rb_ref[...]
    iota = lax.broadcasted_iota(jnp.int32, logits.shape, 1)
    m1 = jnp.max(logits, axis=1, keepdims=True)
    idx1 = jnp.min(jnp.where(logits >= m1, iota, NUM_EXPERTS),
                   axis=1, keepdims=True)
    oh1 = (iota == idx1).astype(jnp.float32)
    masked = jnp.where(iota == idx1, -1e30, logits)
    m2 = jnp.max(masked, axis=1, keepdims=True)
    idx2 = jnp.min(jnp.where(masked >= m2, iota, NUM_EXPERTS),
                   axis=1, keepdims=True)
    oh2 = (iota == idx2).astype(jnp.float32)
    g1 = 1.0 / (1.0 + jnp.exp(m2 - m1))
    g2 = 1.0 - g1

    cnt01 = oh1 + oh2
    counts = jnp.sum(cnt01, axis=0, keepdims=True)
    nblk = jnp.floor((counts + (BLK - 1)) * (1.0 / BLK))
    ue = lax.broadcasted_iota(jnp.int32, (NUM_EXPERTS, NUM_EXPERTS), 0)
    uc = lax.broadcasted_iota(jnp.int32, (NUM_EXPERTS, NUM_EXPERTS), 1)
    upper = (ue < uc).astype(jnp.float32)
    slotbase = BLK * jnp.dot(nblk, upper,
                             preferred_element_type=jnp.float32)

    cnt_bf = cnt01.astype(jnp.bfloat16)
    parts = []
    for b in range(NBAND):
        ri = lax.broadcasted_iota(jnp.int32, (BAND, T), 0) + (BAND * b)
        ci = lax.broadcasted_iota(jnp.int32, (BAND, T), 1)
        lband = (ci < ri).astype(jnp.bfloat16)
        parts.append(jnp.dot(lband, cnt_bf,
                             preferred_element_type=jnp.float32))
    prefix = jnp.concatenate(parts, axis=0)

    rank1 = jnp.sum(oh1 * prefix, axis=1, keepdims=True)
    rank2 = jnp.sum(oh2 * prefix, axis=1, keepdims=True)
    base1 = jnp.sum(oh1 * slotbase, axis=1, keepdims=True)
    base2 = jnp.sum(oh2 * slotbase, axis=1, keepdims=True)
    d1_ref[...] = (base1 + rank1).astype(jnp.int32)
    d2_ref[...] = (base2 + rank2).astype(jnp.int32)
    g1_ref[...] = g1
    g2_ref[...] = g2
    coeff = oh1 * g1 + oh2 * g2
    bias_ref[...] = jnp.dot(coeff, eb_ref[...],
                            preferred_element_type=jnp.float32)
    cnt_ref[...] = counts.astype(jnp.int32)


def _route_call(x, router_w, router_b, expert_b):
    rb2 = router_b.reshape(1, NUM_EXPERTS)
    outs = pl.pallas_call(
        _routing_kernel,
        out_shape=(
            jax.ShapeDtypeStruct((T, 1), jnp.int32),
            jax.ShapeDtypeStruct((T, 1), jnp.int32),
            jax.ShapeDtypeStruct((T, 1), jnp.float32),
            jax.ShapeDtypeStruct((T, 1), jnp.float32),
            jax.ShapeDtypeStruct((T, D_MODEL), jnp.float32),
            jax.ShapeDtypeStruct((1, NUM_EXPERTS), jnp.int32),
        ),
    )(x, router_w, rb2, expert_b)
    d1, d2, g1, g2, bias, cnts = outs
    return (d1.reshape(T), d2.reshape(T), g1.reshape(T), g2.reshape(T),
            bias, cnts.reshape(NUM_EXPERTS))


def _dispatch_body(x_hbm, d1_hbm, d2_hbm, g1_hbm, g2_hbm,
                   xpad_hbm, xv, rows, idxv, gv, sem):
    wid = lax.axis_index("s") * 2 + lax.axis_index("c")
    base = wid * TPW
    pltpu.sync_copy(x_hbm.at[pl.ds(base, TPW)], xv)
    for d_hbm, g_hbm in ((d1_hbm, g1_hbm), (d2_hbm, g2_hbm)):
        pltpu.sync_copy(d_hbm.at[pl.ds(base, TPW)], idxv)
        pltpu.sync_copy(g_hbm.at[pl.ds(base, TPW)], gv)

        def body(j, carry):
            gvec = gv[pl.ds(LANES * j, LANES)]
            for lane in range(LANES):
                gb = gvec[lane]
                tok = LANES * j + lane
                for c in range(D_MODEL // LANES):
                    sl = pl.ds(LANES * c, LANES)
                    rows[tok, sl] = xv[tok, sl] * gb
            return carry

        lax.fori_loop(0, TPW // LANES, body, 0)
        pltpu.async_copy(rows, xpad_hbm.at[idxv], sem).wait()


@functools.cache
def _dispatch_kernel():
    mesh = plsc.VectorSubcoreMesh(core_axis_name="c", subcore_axis_name="s")
    return pl.kernel(
        _dispatch_body,
        mesh=mesh,
        out_type=jax.ShapeDtypeStruct((XPAD, D_MODEL), jnp.float32),
        scratch_types=[
            pltpu.VMEM((TPW, D_MODEL), jnp.float32),
            pltpu.VMEM((TPW, D_MODEL), jnp.float32),
            pltpu.VMEM((TPW,), jnp.int32),
            pltpu.VMEM((TPW,), jnp.float32),
            pltpu.SemaphoreType.DMA,
        ],
    )


def _gmm_kernel(eobl_ref, xp_ref, w_ref, y_ref):
    y_ref[...] = jnp.dot(xp_ref[...].astype(jnp.bfloat16),
                         w_ref[0].astype(jnp.bfloat16),
                         preferred_element_type=jnp.float32)


def _gmm_call(eobl, xpad, expert_w):
    return pl.pallas_call(
        _gmm_kernel,
        grid_spec=pltpu.PrefetchScalarGridSpec(
            num_scalar_prefetch=1,
            grid=(NV,),
            in_specs=[
                pl.BlockSpec((BLK, D_MODEL), lambda v, eobl: (v, 0)),
                pl.BlockSpec((1, D_MODEL, D_MODEL),
                             lambda v, eobl: (eobl[v], 0, 0)),
            ],
            out_specs=pl.BlockSpec((BLK, D_MODEL), lambda v, eobl: (v, 0)),
        ),
        out_shape=jax.ShapeDtypeStruct((XPAD, D_MODEL), jnp.float32),
        compiler_params=pltpu.CompilerParams(
            dimension_semantics=("arbitrary",)),
    )(eobl, xpad, expert_w)


_HTPW = TPW // 2


def _combine_body(y_hbm, d1_hbm, d2_hbm, bias_hbm,
                  out_hbm, i1, i2, ya, yb, ov, sem):
    wid = lax.axis_index("s") * 2 + lax.axis_index("c")
    for h in range(2):
        hbase = wid * TPW + _HTPW * h
        pltpu.sync_copy(d1_hbm.at[pl.ds(hbase, _HTPW)], i1)
        pltpu.sync_copy(d2_hbm.at[pl.ds(hbase, _HTPW)], i2)
        pltpu.async_copy(y_hbm.at[i1], ya, sem).wait()
        pltpu.async_copy(y_hbm.at[i2], yb, sem).wait()
        pltpu.sync_copy(bias_hbm.at[pl.ds(hbase, _HTPW)], ov)

        def body(i, carry):
            for c in range(D_MODEL // LANES):
                sl = pl.ds(LANES * c, LANES)
                ov[i, sl] = ov[i, sl] + ya[i, sl] + yb[i, sl]
            return carry

        lax.fori_loop(0, _HTPW, body, 0)
        pltpu.sync_copy(ov, out_hbm.at[pl.ds(hbase, _HTPW)])


@functools.cache
def _combine_kernel():
    mesh = plsc.VectorSubcoreMesh(core_axis_name="c", subcore_axis_name="s")
    return pl.kernel(
        _combine_body,
        mesh=mesh,
        out_type=jax.ShapeDtypeStruct((T, D_MODEL), jnp.float32),
        scratch_types=[
            pltpu.VMEM((_HTPW,), jnp.int32),
            pltpu.VMEM((_HTPW,), jnp.int32),
            pltpu.VMEM((_HTPW, D_MODEL), jnp.float32),
            pltpu.VMEM((_HTPW, D_MODEL), jnp.float32),
            pltpu.VMEM((_HTPW, D_MODEL), jnp.float32),
            pltpu.SemaphoreType.DMA,
        ],
    )


@jax.jit
def kernel(x, router_w, router_b, expert_w, expert_b):
    d1, d2, g1, g2, bias, counts = _route_call(x, router_w, router_b,
                                               expert_b)
    nblk = (counts + (BLK - 1)) // BLK
    eobl = jnp.repeat(jnp.arange(NUM_EXPERTS, dtype=jnp.int32), nblk,
                      total_repeat_length=NV)
    xpad = _dispatch_kernel()(x, d1, d2, g1, g2)
    y = _gmm_call(eobl, xpad, expert_w)
    return _combine_kernel()(y, d1, d2, bias)
